# baseline (speedup 1.0000x reference)
import functools

import jax
import jax.numpy as jnp
from jax.experimental import pallas as pl


def _chamfer_body(a1_ref, a2_ref, dist1_ref, dist2_ref):
    a1aug = a1_ref[0]
    a2aug = a2_ref[0]
    d = jax.lax.dot_general(
        a1aug, a2aug,
        dimension_numbers=(((0,), (0,)), ((), ())),
        preferred_element_type=jnp.float32,
    )
    dist1_ref[0, 0] = jnp.min(d, axis=1)
    colmin = jnp.min(d, axis=0)

    @pl.when(pl.program_id(1) == 0)
    def _init():
        dist2_ref[0, 0] = colmin

    @pl.when(pl.program_id(1) != 0)
    def _acc():
        dist2_ref[0, 0] = jnp.minimum(dist2_ref[0, 0], colmin)


@functools.partial(jax.jit, static_argnames=("block_n", "interpret"))
def _chamfer(array1, array2, block_n=1024, interpret=False):
    b, n, _ = array1.shape
    m = array2.shape[1]
    a1t = array1.transpose(0, 2, 1)
    a2t = array2.transpose(0, 2, 1)
    x2 = jnp.sum(a1t * a1t, axis=1, keepdims=True)
    y2 = jnp.sum(a2t * a2t, axis=1, keepdims=True)
    ones_n = jnp.ones((b, 1, n), jnp.float32)
    ones_m = jnp.ones((b, 1, m), jnp.float32)
    zeros_n = jnp.zeros((b, 3, n), jnp.float32)
    zeros_m = jnp.zeros((b, 3, m), jnp.float32)
    a1aug = jnp.concatenate([a1t, x2, ones_n, zeros_n], axis=1)
    a2aug = jnp.concatenate([-2.0 * a2t, ones_m, y2, zeros_m], axis=1)

    nb = n // block_n
    grid = (b, nb)
    dist1, dist2 = pl.pallas_call(
        _chamfer_body,
        grid=grid,
        in_specs=[
            pl.BlockSpec((1, 8, block_n), lambda i, j: (i, 0, j)),
            pl.BlockSpec((1, 8, m), lambda i, j: (i, 0, 0)),
        ],
        out_specs=[
            pl.BlockSpec((1, 1, block_n), lambda i, j: (i * nb + j, 0, 0)),
            pl.BlockSpec((1, 1, m), lambda i, j: (i, 0, 0)),
        ],
        out_shape=[
            jax.ShapeDtypeStruct((b * nb, 1, block_n), jnp.float32),
            jax.ShapeDtypeStruct((b, 1, m), jnp.float32),
        ],
        interpret=interpret,
    )(a1aug, a2aug)
    return jnp.mean(dist1) + jnp.mean(dist2)


def kernel(array1, array2):
    return _chamfer(array1, array2)


# speedup vs baseline: 1.4880x; 1.4880x over previous
import functools

import jax
import jax.numpy as jnp
from jax.experimental import pallas as pl
from jax.experimental.pallas import tpu as pltpu


def _chamfer_body(a1_ref, a2_ref, out_ref):
    b = pl.num_programs(0)
    a1 = a1_ref[0]
    a2 = a2_ref[0]
    n = a1.shape[0]
    m = a2.shape[0]
    x2 = jnp.sum(a1 * a1, axis=1, keepdims=True)
    y2 = jnp.sum(a2 * a2, axis=1, keepdims=True)
    ones_n = jnp.ones((n, 1), dtype=a1.dtype)
    ones_m = jnp.ones((m, 1), dtype=a2.dtype)
    lhs = jnp.concatenate([a1, x2, ones_n], axis=1)
    rhs = jnp.concatenate([-2.0 * a2, ones_m, y2], axis=1)
    d = jax.lax.dot_general(
        lhs, rhs,
        dimension_numbers=(((1,), (1,)), ((), ())),
        preferred_element_type=jnp.float32,
    )
    s1 = jnp.sum(jnp.min(d, axis=1)) * (1.0 / (b * n))
    s2 = jnp.sum(jnp.min(d, axis=0)) * (1.0 / (b * m))

    @pl.when(pl.program_id(0) == 0)
    def _init():
        out_ref[0, 0] = s1 + s2

    @pl.when(pl.program_id(0) != 0)
    def _acc():
        out_ref[0, 0] += s1 + s2


@functools.partial(jax.jit, static_argnames=("interpret",))
def _chamfer(array1, array2, interpret=False):
    b, n, _ = array1.shape
    m = array2.shape[1]
    out = pl.pallas_call(
        _chamfer_body,
        grid=(b,),
        in_specs=[
            pl.BlockSpec((1, n, 3), lambda i: (i, 0, 0)),
            pl.BlockSpec((1, m, 3), lambda i: (i, 0, 0)),
        ],
        out_specs=pl.BlockSpec(
            (1, 1), lambda i: (0, 0), memory_space=pltpu.SMEM
        ),
        out_shape=jax.ShapeDtypeStruct((1, 1), jnp.float32),
        interpret=interpret,
    )(array1, array2)
    return out.reshape(())


def kernel(array1, array2):
    return _chamfer(array1, array2)


# speedup vs baseline: 2.1578x; 1.4501x over previous
import functools

import jax
import jax.numpy as jnp
from jax.experimental import pallas as pl
from jax.experimental.pallas import tpu as pltpu


def _chamfer_body(a1_ref, a2_ref, out_ref, *, batches):
    i = pl.program_id(0)
    b = batches

    def rows5(ref, scale):
        x = ref[pl.ds(i, 1), :]
        y = ref[pl.ds(i + b, 1), :]
        z = ref[pl.ds(i + 2 * b, 1), :]
        sq = x * x + y * y + z * z
        ones = jnp.ones_like(x)
        if scale is None:
            return jnp.concatenate([x, y, z, sq, ones], axis=0)
        return jnp.concatenate(
            [scale * x, scale * y, scale * z, ones, sq], axis=0
        )

    lhs = rows5(a1_ref, None)
    rhs = rows5(a2_ref, -2.0)
    d = jax.lax.dot_general(
        lhs, rhs,
        dimension_numbers=(((0,), (0,)), ((), ())),
        preferred_element_type=jnp.float32,
    )
    n = d.shape[0]
    m = d.shape[1]
    s1 = jnp.sum(jnp.min(d, axis=1)) * (1.0 / (b * n))
    s2 = jnp.sum(jnp.min(d, axis=0)) * (1.0 / (b * m))

    @pl.when(i == 0)
    def _init():
        out_ref[0, 0] = s1 + s2

    @pl.when(i != 0)
    def _acc():
        out_ref[0, 0] += s1 + s2


@functools.partial(jax.jit, static_argnames=("interpret",))
def _chamfer(array1, array2, interpret=False):
    b, n, _ = array1.shape
    m = array2.shape[1]
    a1v = array1.transpose(2, 0, 1).reshape(3 * b, n)
    a2v = array2.transpose(2, 0, 1).reshape(3 * b, m)
    out = pl.pallas_call(
        functools.partial(_chamfer_body, batches=b),
        grid=(b,),
        in_specs=[
            pl.BlockSpec((3 * b, n), lambda i: (0, 0)),
            pl.BlockSpec((3 * b, m), lambda i: (0, 0)),
        ],
        out_specs=pl.BlockSpec(
            (1, 1), lambda i: (0, 0), memory_space=pltpu.SMEM
        ),
        out_shape=jax.ShapeDtypeStruct((1, 1), jnp.float32),
        interpret=interpret,
    )(a1v, a2v)
    return out.reshape(())


def kernel(array1, array2):
    return _chamfer(array1, array2)


# speedup vs baseline: 2.2927x; 1.0625x over previous
import functools

import jax
import jax.numpy as jnp
from jax.experimental import pallas as pl
from jax.experimental.pallas import tpu as pltpu


def _tree_min(parts):
    while len(parts) > 1:
        nxt = [jnp.minimum(parts[k], parts[k + 1]) for k in range(0, len(parts) - 1, 2)]
        if len(parts) % 2:
            nxt.append(parts[-1])
        parts = nxt
    return parts[0]


def _chamfer_body(a1_ref, a2_ref, out_ref, pmin1_ref, pmin2_ref, *, batches):
    i = pl.program_id(0)
    b = batches

    def tail_sums():
        pm1 = pmin1_ref[...]
        pm2 = pmin2_ref[...]
        n = pm1.shape[0]
        m = pm2.shape[1]
        s1 = jnp.sum(jnp.min(pm1, axis=1)) * (1.0 / (b * n))
        s2 = jnp.sum(jnp.min(pm2, axis=0)) * (1.0 / (b * m))
        return s1 + s2

    contrib = jnp.where(i > 0, tail_sums(), 0.0)
    acc = jnp.where(i > 0, out_ref[0, 0], 0.0)
    out_ref[0, 0] = acc + contrib

    def rows5(ref, scale):
        x = ref[pl.ds(i, 1), :]
        y = ref[pl.ds(i + b, 1), :]
        z = ref[pl.ds(i + 2 * b, 1), :]
        sq = x * x + y * y + z * z
        ones = jnp.ones_like(x)
        if scale is None:
            return jnp.concatenate([x, y, z, sq, ones], axis=0)
        return jnp.concatenate(
            [scale * x, scale * y, scale * z, ones, sq], axis=0
        )

    lhs = rows5(a1_ref, None)
    rhs = rows5(a2_ref, -2.0)
    d = jax.lax.dot_general(
        lhs, rhs,
        dimension_numbers=(((0,), (0,)), ((), ())),
        preferred_element_type=jnp.float32,
    )
    m = d.shape[1]
    pmin1_ref[...] = _tree_min(
        [d[:, k * 128:(k + 1) * 128] for k in range(m // 128)]
    )
    pmin2_ref[...] = _tree_min(
        [d[k * 8:(k + 1) * 8, :] for k in range(d.shape[0] // 8)]
    )

    @pl.when(i == b - 1)
    def _last():
        out_ref[0, 0] += tail_sums()


@functools.partial(jax.jit, static_argnames=("interpret",))
def _chamfer(array1, array2, interpret=False):
    b, n, _ = array1.shape
    m = array2.shape[1]
    a1v = array1.transpose(2, 0, 1).reshape(3 * b, n)
    a2v = array2.transpose(2, 0, 1).reshape(3 * b, m)
    out = pl.pallas_call(
        functools.partial(_chamfer_body, batches=b),
        grid=(b,),
        in_specs=[
            pl.BlockSpec((3 * b, n), lambda i: (0, 0)),
            pl.BlockSpec((3 * b, m), lambda i: (0, 0)),
        ],
        out_specs=pl.BlockSpec(
            (1, 1), lambda i: (0, 0), memory_space=pltpu.SMEM
        ),
        out_shape=jax.ShapeDtypeStruct((1, 1), jnp.float32),
        scratch_shapes=[
            pltpu.VMEM((n, 128), jnp.float32),
            pltpu.VMEM((8, m), jnp.float32),
        ],
        interpret=interpret,
    )(a1v, a2v)
    return out.reshape(())


def kernel(array1, array2):
    return _chamfer(array1, array2)


# speedup vs baseline: 2.3907x; 1.0427x over previous
import functools

import jax
import jax.numpy as jnp
from jax.experimental import pallas as pl
from jax.experimental.pallas import tpu as pltpu

_UNROLL = 2


def _tree_min(parts):
    while len(parts) > 1:
        nxt = [jnp.minimum(parts[k], parts[k + 1]) for k in range(0, len(parts) - 1, 2)]
        if len(parts) % 2:
            nxt.append(parts[-1])
        parts = nxt
    return parts[0]


def _chamfer_body(a1_ref, a2_ref, out_ref, pmin1_ref, pmin2_ref, *, batches):
    i = pl.program_id(0)
    b = batches
    u = _UNROLL

    def tail_sums():
        total = 0.0
        for k in range(u):
            pm1 = pmin1_ref[k]
            pm2 = pmin2_ref[k]
            n = pm1.shape[0]
            m = pm2.shape[1]
            s1 = jnp.sum(jnp.min(pm1, axis=1)) * (1.0 / (b * n))
            s2 = jnp.sum(jnp.min(pm2, axis=0)) * (1.0 / (b * m))
            total = total + (s1 + s2)
        return total

    contrib = jnp.where(i > 0, tail_sums(), 0.0)
    acc = jnp.where(i > 0, out_ref[0, 0], 0.0)
    out_ref[0, 0] = acc + contrib

    def rows5(ref, bi, scale):
        x = ref[pl.ds(bi, 1), :]
        y = ref[pl.ds(bi + b, 1), :]
        z = ref[pl.ds(bi + 2 * b, 1), :]
        sq = x * x + y * y + z * z
        ones = jnp.ones_like(x)
        if scale is None:
            return jnp.concatenate([x, y, z, sq, ones], axis=0)
        return jnp.concatenate(
            [scale * x, scale * y, scale * z, ones, sq], axis=0
        )

    for k in range(u):
        bi = i * u + k
        lhs = rows5(a1_ref, bi, None)
        rhs = rows5(a2_ref, bi, -2.0)
        d = jax.lax.dot_general(
            lhs, rhs,
            dimension_numbers=(((0,), (0,)), ((), ())),
            preferred_element_type=jnp.float32,
        )
        m = d.shape[1]
        pmin1_ref[k] = _tree_min(
            [d[:, c * 128:(c + 1) * 128] for c in range(m // 128)]
        )
        pmin2_ref[k] = _tree_min(
            [d[r * 8:(r + 1) * 8, :] for r in range(d.shape[0] // 8)]
        )

    @pl.when(i == (b // u) - 1)
    def _last():
        out_ref[0, 0] += tail_sums()


@functools.partial(jax.jit, static_argnames=("interpret",))
def _chamfer(array1, array2, interpret=False):
    b, n, _ = array1.shape
    m = array2.shape[1]
    a1v = array1.transpose(2, 0, 1).reshape(3 * b, n)
    a2v = array2.transpose(2, 0, 1).reshape(3 * b, m)
    out = pl.pallas_call(
        functools.partial(_chamfer_body, batches=b),
        grid=(b // _UNROLL,),
        in_specs=[
            pl.BlockSpec((3 * b, n), lambda i: (0, 0)),
            pl.BlockSpec((3 * b, m), lambda i: (0, 0)),
        ],
        out_specs=pl.BlockSpec(
            (1, 1), lambda i: (0, 0), memory_space=pltpu.SMEM
        ),
        out_shape=jax.ShapeDtypeStruct((1, 1), jnp.float32),
        scratch_shapes=[
            pltpu.VMEM((_UNROLL, n, 128), jnp.float32),
            pltpu.VMEM((_UNROLL, 8, m), jnp.float32),
        ],
        interpret=interpret,
    )(a1v, a2v)
    return out.reshape(())


def kernel(array1, array2):
    return _chamfer(array1, array2)


# speedup vs baseline: 2.4041x; 1.0056x over previous
import functools

import jax
import jax.numpy as jnp
from jax.experimental import pallas as pl
from jax.experimental.pallas import tpu as pltpu

_UNROLL = 4


def _tree_min(parts):
    while len(parts) > 1:
        nxt = [jnp.minimum(parts[k], parts[k + 1]) for k in range(0, len(parts) - 1, 2)]
        if len(parts) % 2:
            nxt.append(parts[-1])
        parts = nxt
    return parts[0]


def _chamfer_body(a1_ref, a2_ref, out_ref, pmin1_ref, pmin2_ref, *, batches):
    i = pl.program_id(0)
    b = batches
    u = _UNROLL

    def tail_sums():
        total = 0.0
        for k in range(u):
            pm1 = pmin1_ref[k]
            pm2 = pmin2_ref[k]
            n = pm1.shape[0]
            m = pm2.shape[1]
            s1 = jnp.sum(jnp.min(pm1, axis=1)) * (1.0 / (b * n))
            s2 = jnp.sum(jnp.min(pm2, axis=0)) * (1.0 / (b * m))
            total = total + (s1 + s2)
        return total

    contrib = jnp.where(i > 0, tail_sums(), 0.0)
    acc = jnp.where(i > 0, out_ref[0, 0], 0.0)
    out_ref[0, 0] = acc + contrib

    def rows5(ref, bi, scale):
        x = ref[pl.ds(bi, 1), :]
        y = ref[pl.ds(bi + b, 1), :]
        z = ref[pl.ds(bi + 2 * b, 1), :]
        sq = x * x + y * y + z * z
        ones = jnp.ones_like(x)
        if scale is None:
            return jnp.concatenate([x, y, z, sq, ones], axis=0)
        return jnp.concatenate(
            [scale * x, scale * y, scale * z, ones, sq], axis=0
        )

    for k in range(u):
        bi = i * u + k
        lhs = rows5(a1_ref, bi, None)
        rhs = rows5(a2_ref, bi, -2.0)
        d = jax.lax.dot_general(
            lhs, rhs,
            dimension_numbers=(((0,), (0,)), ((), ())),
            preferred_element_type=jnp.float32,
        )
        m = d.shape[1]
        pmin1_ref[k] = _tree_min(
            [d[:, c * 128:(c + 1) * 128] for c in range(m // 128)]
        )
        pmin2_ref[k] = _tree_min(
            [d[r * 8:(r + 1) * 8, :] for r in range(d.shape[0] // 8)]
        )

    @pl.when(i == (b // u) - 1)
    def _last():
        out_ref[0, 0] += tail_sums()


@functools.partial(jax.jit, static_argnames=("interpret",))
def _chamfer(array1, array2, interpret=False):
    b, n, _ = array1.shape
    m = array2.shape[1]
    a1v = array1.transpose(2, 0, 1).reshape(3 * b, n)
    a2v = array2.transpose(2, 0, 1).reshape(3 * b, m)
    out = pl.pallas_call(
        functools.partial(_chamfer_body, batches=b),
        grid=(b // _UNROLL,),
        in_specs=[
            pl.BlockSpec((3 * b, n), lambda i: (0, 0)),
            pl.BlockSpec((3 * b, m), lambda i: (0, 0)),
        ],
        out_specs=pl.BlockSpec(
            (1, 1), lambda i: (0, 0), memory_space=pltpu.SMEM
        ),
        out_shape=jax.ShapeDtypeStruct((1, 1), jnp.float32),
        scratch_shapes=[
            pltpu.VMEM((_UNROLL, n, 128), jnp.float32),
            pltpu.VMEM((_UNROLL, 8, m), jnp.float32),
        ],
        interpret=interpret,
    )(a1v, a2v)
    return out.reshape(())


def kernel(array1, array2):
    return _chamfer(array1, array2)
